# baseline (device time: 22121 ns/iter reference)
import jax
import jax.numpy as jnp
from jax import lax
from jax.experimental import pallas as pl
from jax.experimental.pallas import tpu as pltpu

N_DEV = 16
NZ = 4
NQ = 4
N_SEND = 32

_OFFS = (0, -1, 1, -2, 2, -3, 3)


def kernel(x, w_mat):
    m_per, k = x.shape
    _, n_per = w_mat.shape

    def body(x_ref, w_ref, out_ref, xfull_ref, src_ref,
             send_sems, zcol_sems, ipx_sems, ipy_sems, dvx_sems, dvy_sems,
             copy_sem):
        my = lax.axis_index("i")
        z = my // NQ
        q = my % NQ
        x_nbr = NQ * z + (q ^ 1)
        y_nbr = NQ * z + (3 - q)

        src_ref[...] = x_ref[...].astype(jnp.bfloat16)
        wb = w_ref[...].astype(jnp.bfloat16)

        barrier = pltpu.get_barrier_semaphore()
        for tgt in (x_nbr, y_nbr):
            pl.semaphore_signal(barrier, inc=1, device_id=(tgt,),
                                device_id_type=pl.DeviceIdType.MESH)
        for zo in range(NZ):
            pl.semaphore_signal(barrier, inc=1, device_id=(NQ * zo + q,),
                                device_id_type=pl.DeviceIdType.MESH)
        pl.semaphore_wait(barrier, 6)

        own = pltpu.make_async_copy(src_ref, xfull_ref.at[my], copy_sem)
        own.start()
        own.wait()

        sends = []
        sidx = [0]

        def send_chunk(slot, tgt, rsem, cond):
            i = sidx[0]
            sidx[0] += 1
            rdma = pltpu.make_async_remote_copy(
                src_ref=xfull_ref.at[slot],
                dst_ref=xfull_ref.at[slot],
                send_sem=send_sems.at[i],
                recv_sem=rsem,
                device_id=(tgt,),
                device_id_type=pl.DeviceIdType.MESH,
            )
            if cond is None:
                rdma.start()
            else:
                @pl.when(cond)
                def _():
                    rdma.start()
            sends.append((cond, rdma))

        def wait_chunk(slot, rsem, cond):
            recv = pltpu.make_async_remote_copy(
                src_ref=src_ref,
                dst_ref=xfull_ref.at[slot],
                send_sem=send_sems.at[0],
                recv_sem=rsem,
                device_id=(my,),
                device_id_type=pl.DeviceIdType.MESH,
            )
            if cond is None:
                recv.wait_recv()
            else:
                @pl.when(cond)
                def _():
                    recv.wait_recv()

        valid = []
        ranks = []
        racc = jnp.int32(-1)
        for off in _OFFS:
            zo_t = z + off
            v = jnp.logical_and(zo_t >= 0, zo_t <= NZ - 1)
            racc = racc + v.astype(jnp.int32)
            valid.append(v)
            ranks.append(racc)

        for zo in range(NZ):
            send_chunk(my, NQ * zo + q, zcol_sems.at[z], z != zo)

        send_chunk(my, x_nbr, ipx_sems.at[z], None)
        send_chunk(my, y_nbr, ipy_sems.at[z], None)

        for i, off in enumerate(_OFFS):
            if off == 0:
                continue
            cond = valid[i]
            zo = jnp.clip(z + off, 0, NZ - 1)
            slot = NQ * zo + q
            wait_chunk(slot, zcol_sems.at[zo], cond)
            send_chunk(slot, x_nbr, ipx_sems.at[zo], cond)
            send_chunk(slot, y_nbr, ipy_sems.at[zo], cond)

        for i, off in enumerate(_OFFS):
            cond = valid[i]
            r = jnp.clip(ranks[i], 0, NZ - 1)
            r_even = (r % 2) == 0
            rhalf = jnp.clip(r // 2, 0, 1)
            zo = jnp.clip(z + off, 0, NZ - 1)

            slot_x = NQ * zo + (q ^ 1)
            wait_chunk(slot_x, ipx_sems.at[zo], cond)
            send_chunk(slot_x, y_nbr, dvy_sems.at[rhalf],
                       jnp.logical_and(cond, r_even))

            slot_y = NQ * zo + (3 - q)
            wait_chunk(slot_y, ipy_sems.at[zo], cond)
            send_chunk(slot_y, x_nbr, dvx_sems.at[rhalf],
                       jnp.logical_and(cond, jnp.logical_not(r_even)))

        for i, off in enumerate(_OFFS):
            cond = valid[i]
            r = jnp.clip(ranks[i], 0, NZ - 1)
            r_even = (r % 2) == 0
            rhalf = jnp.clip(r // 2, 0, 1)
            zo = jnp.clip(z + off, 0, NZ - 1)
            slot_d = NQ * zo + (q ^ 2)
            wait_chunk(slot_d, dvy_sems.at[rhalf],
                       jnp.logical_and(cond, r_even))
            wait_chunk(slot_d, dvx_sems.at[rhalf],
                       jnp.logical_and(cond, jnp.logical_not(r_even)))

            @pl.when(cond)
            def _(zo=zo):
                xp = xfull_ref[pl.ds(NQ * zo, NQ)].reshape(NQ * m_per, k)
                out_ref[pl.ds(zo * NQ * m_per, NQ * m_per), :] = jnp.dot(
                    xp, wb, preferred_element_type=jnp.float32)

        for cond, rdma in sends:
            if cond is None:
                rdma.wait_send()
            else:
                @pl.when(cond)
                def _(rdma=rdma):
                    rdma.wait_send()

        assert sidx[0] == N_SEND, sidx[0]

    return pl.pallas_call(
        body,
        out_shape=jax.ShapeDtypeStruct((N_DEV * m_per, n_per), jnp.float32),
        in_specs=[
            pl.BlockSpec(memory_space=pltpu.VMEM),
            pl.BlockSpec(memory_space=pltpu.VMEM),
        ],
        out_specs=pl.BlockSpec(memory_space=pltpu.VMEM),
        scratch_shapes=[
            pltpu.VMEM((N_DEV, m_per, k), jnp.bfloat16),
            pltpu.VMEM((m_per, k), jnp.bfloat16),
            pltpu.SemaphoreType.DMA((N_SEND,)),
            pltpu.SemaphoreType.DMA((NZ,)),
            pltpu.SemaphoreType.DMA((NZ,)),
            pltpu.SemaphoreType.DMA((NZ,)),
            pltpu.SemaphoreType.DMA((2,)),
            pltpu.SemaphoreType.DMA((2,)),
            pltpu.SemaphoreType.DMA,
        ],
        compiler_params=pltpu.CompilerParams(collective_id=0),
    )(x, w_mat)


# device time: 22005 ns/iter; 1.0053x vs baseline; 1.0053x over previous
import jax
import jax.numpy as jnp
from jax import lax
from jax.experimental import pallas as pl
from jax.experimental.pallas import tpu as pltpu

N_DEV = 16
NZ = 4
NQ = 4
N_SEND = 32

_OFFS = (0, -1, 1, -2, 2, -3, 3)


def kernel(x, w_mat):
    m_per, k = x.shape
    _, n_per = w_mat.shape

    def body(x_ref, w_ref, out_ref, xfull_ref, src_ref,
             send_sems, zcol_sems, ipx_sems, ipy_sems, dvx_sems, dvy_sems,
             copy_sem):
        my = lax.axis_index("i")
        z = my // NQ
        q = my % NQ
        x_nbr = NQ * z + (q ^ 1)
        y_nbr = NQ * z + (3 - q)

        src_ref[...] = x_ref[...].astype(jnp.bfloat16)
        wb = w_ref[...].astype(jnp.bfloat16)

        barrier = pltpu.get_barrier_semaphore()
        for tgt in (x_nbr, y_nbr):
            pl.semaphore_signal(barrier, inc=1, device_id=(tgt,),
                                device_id_type=pl.DeviceIdType.MESH)
        for zo in range(NZ):
            pl.semaphore_signal(barrier, inc=1, device_id=(NQ * zo + q,),
                                device_id_type=pl.DeviceIdType.MESH)

        own = pltpu.make_async_copy(src_ref, xfull_ref.at[my], copy_sem)
        own.start()
        pl.semaphore_wait(barrier, 6)
        own.wait()

        sends = []
        sidx = [0]

        def send_chunk(slot, tgt, rsem, cond):
            i = sidx[0]
            sidx[0] += 1
            rdma = pltpu.make_async_remote_copy(
                src_ref=xfull_ref.at[slot],
                dst_ref=xfull_ref.at[slot],
                send_sem=send_sems.at[i],
                recv_sem=rsem,
                device_id=(tgt,),
                device_id_type=pl.DeviceIdType.MESH,
            )
            if cond is None:
                rdma.start()
            else:
                @pl.when(cond)
                def _():
                    rdma.start()
            sends.append((cond, rdma))

        def wait_chunk(slot, rsem, cond):
            recv = pltpu.make_async_remote_copy(
                src_ref=src_ref,
                dst_ref=xfull_ref.at[slot],
                send_sem=send_sems.at[0],
                recv_sem=rsem,
                device_id=(my,),
                device_id_type=pl.DeviceIdType.MESH,
            )
            if cond is None:
                recv.wait_recv()
            else:
                @pl.when(cond)
                def _():
                    recv.wait_recv()

        valid = []
        ranks = []
        racc = jnp.int32(-1)
        for off in _OFFS:
            zo_t = z + off
            v = jnp.logical_and(zo_t >= 0, zo_t <= NZ - 1)
            racc = racc + v.astype(jnp.int32)
            valid.append(v)
            ranks.append(racc)

        for zo in range(NZ):
            send_chunk(my, NQ * zo + q, zcol_sems.at[z], z != zo)

        send_chunk(my, x_nbr, ipx_sems.at[z], None)
        send_chunk(my, y_nbr, ipy_sems.at[z], None)

        for i, off in enumerate(_OFFS):
            if off == 0:
                continue
            cond = valid[i]
            zo = jnp.clip(z + off, 0, NZ - 1)
            slot = NQ * zo + q
            wait_chunk(slot, zcol_sems.at[zo], cond)
            send_chunk(slot, x_nbr, ipx_sems.at[zo], cond)
            send_chunk(slot, y_nbr, ipy_sems.at[zo], cond)

        for i, off in enumerate(_OFFS):
            cond = valid[i]
            r = jnp.clip(ranks[i], 0, NZ - 1)
            r_even = (r % 2) == 0
            rhalf = jnp.clip(r // 2, 0, 1)
            zo = jnp.clip(z + off, 0, NZ - 1)

            slot_x = NQ * zo + (q ^ 1)
            wait_chunk(slot_x, ipx_sems.at[zo], cond)
            send_chunk(slot_x, y_nbr, dvy_sems.at[rhalf],
                       jnp.logical_and(cond, r_even))

            slot_y = NQ * zo + (3 - q)
            wait_chunk(slot_y, ipy_sems.at[zo], cond)
            send_chunk(slot_y, x_nbr, dvx_sems.at[rhalf],
                       jnp.logical_and(cond, jnp.logical_not(r_even)))

        for i, off in enumerate(_OFFS):
            cond = valid[i]
            r = jnp.clip(ranks[i], 0, NZ - 1)
            r_even = (r % 2) == 0
            rhalf = jnp.clip(r // 2, 0, 1)
            zo = jnp.clip(z + off, 0, NZ - 1)
            slot_d = NQ * zo + (q ^ 2)
            wait_chunk(slot_d, dvy_sems.at[rhalf],
                       jnp.logical_and(cond, r_even))
            wait_chunk(slot_d, dvx_sems.at[rhalf],
                       jnp.logical_and(cond, jnp.logical_not(r_even)))

            @pl.when(cond)
            def _(zo=zo):
                xp = xfull_ref[pl.ds(NQ * zo, NQ)].reshape(NQ * m_per, k)
                out_ref[pl.ds(zo * NQ * m_per, NQ * m_per), :] = jnp.dot(
                    xp, wb, preferred_element_type=jnp.float32)

        for cond, rdma in sends:
            if cond is None:
                rdma.wait_send()
            else:
                @pl.when(cond)
                def _(rdma=rdma):
                    rdma.wait_send()

        assert sidx[0] == N_SEND, sidx[0]

    return pl.pallas_call(
        body,
        out_shape=jax.ShapeDtypeStruct((N_DEV * m_per, n_per), jnp.float32),
        in_specs=[
            pl.BlockSpec(memory_space=pltpu.VMEM),
            pl.BlockSpec(memory_space=pltpu.VMEM),
        ],
        out_specs=pl.BlockSpec(memory_space=pltpu.VMEM),
        scratch_shapes=[
            pltpu.VMEM((N_DEV, m_per, k), jnp.bfloat16),
            pltpu.VMEM((m_per, k), jnp.bfloat16),
            pltpu.SemaphoreType.DMA((N_SEND,)),
            pltpu.SemaphoreType.DMA((NZ,)),
            pltpu.SemaphoreType.DMA((NZ,)),
            pltpu.SemaphoreType.DMA((NZ,)),
            pltpu.SemaphoreType.DMA((2,)),
            pltpu.SemaphoreType.DMA((2,)),
            pltpu.SemaphoreType.DMA,
        ],
        compiler_params=pltpu.CompilerParams(collective_id=0),
    )(x, w_mat)
